# baseline (speedup 1.0000x reference)
import functools

import jax
import jax.numpy as jnp
from jax import lax
from jax.experimental import pallas as pl
from jax.experimental.pallas import tpu as pltpu
from jax.experimental.pallas import tpu_sc as plsc

NUM_STATES = 12
NPAIR_STATES = NUM_STATES * NUM_STATES
EMBED_DIM = 64
ROW = 2 * EMBED_DIM
N_POS = 1024
N_PP = N_POS // 2
BATCH = 4096
GRP = 8
N_GRP = BATCH // GRP

NC = 2
NS = 16
L = 16
NW = NC * NS
G_PER_W = N_GRP // NW
CHUNK = 128
UNIT = 128
N_UNITS = (N_PP * GRP) // UNIT
T_ROWS = N_PP * NPAIR_STATES
PP_PER_TILE = N_PP // NS

_mesh = plsc.VectorSubcoreMesh(core_axis_name="c", subcore_axis_name="s")


@functools.partial(
    pl.kernel,
    out_type=[
        jax.ShapeDtypeStruct((N_GRP, N_PP * GRP, ROW), jnp.float32),
        jax.ShapeDtypeStruct((NC * T_ROWS, ROW), jnp.float32),
    ],
    mesh=_mesh,
    compiler_params=pltpu.CompilerParams(
        use_tc_tiling_on_sc=False, needs_layout_passes=False
    ),
    scratch_types=[
        pltpu.VMEM((NUM_STATES, EMBED_DIM), jnp.float32),
        pltpu.VMEM((2 * PP_PER_TILE, EMBED_DIM), jnp.float32),
        pltpu.VMEM((2, NPAIR_STATES // 2, ROW), jnp.float32),
        pltpu.VMEM((2 * NUM_STATES, EMBED_DIM), jnp.float32),
        pltpu.VMEM((N_PP * GRP,), jnp.int32),
        pltpu.VMEM((2, GRP, N_POS), jnp.int32),
        pltpu.VMEM((2, N_PP * GRP), jnp.int32),
        pltpu.VMEM((4, UNIT, ROW), jnp.float32),
        pltpu.SemaphoreType.DMA,
        pltpu.SemaphoreType.DMA,
        pltpu.SemaphoreType.DMA,
        pltpu.SemaphoreType.DMA,
        pltpu.SemaphoreType.DMA,
        pltpu.SemaphoreType.DMA,
        pltpu.SemaphoreType.DMA,
        pltpu.SemaphoreType.DMA,
        pltpu.SemaphoreType.DMA,
    ],
)
def _sc_embed(grid_hbm, state_hbm, pos_hbm, out_hbm, t_hbm,
              sbuf, pbuf, bbuf, lrtmp, pvec, ibuf, pibuf, rbuf,
              gsem0, gsem1, gsem2, gsem3, isem, wsem0, wsem1, wsem2, wsem3):
    cid = lax.axis_index("c")
    sid = lax.axis_index("s")
    wid = sid * NC + cid

    pltpu.sync_copy(state_hbm, sbuf)
    pltpu.sync_copy(pos_hbm.at[pl.ds(sid * 2 * PP_PER_TILE, 2 * PP_PER_TILE)], pbuf)

    HSLAB = NPAIR_STATES // 2

    def build_pair(ppl, carry):
        def halves(s, c2):
            for d in range(EMBED_DIM // L):
                lrtmp[s, pl.ds(d * L, L)] = (
                    pbuf[2 * ppl, pl.ds(d * L, L)] + sbuf[s, pl.ds(d * L, L)]
                )
                lrtmp[NUM_STATES + s, pl.ds(d * L, L)] = (
                    pbuf[2 * ppl + 1, pl.ds(d * L, L)] + sbuf[s, pl.ds(d * L, L)]
                )
            return c2

        lax.fori_loop(0, NUM_STATES, halves, 0)

        for q in range(2):
            bwsem = wsem0 if q == 0 else wsem1

            @pl.when(ppl >= 1)
            def _wait_prev_slab():
                pltpu.make_async_copy(
                    bbuf.at[q], t_hbm.at[pl.ds(0, HSLAB)], bwsem
                ).wait()

            def se_body(se6, c2):
                se = se6 + q * (NUM_STATES // 2)
                lvals = [lrtmp[se, pl.ds(d * L, L)] for d in range(EMBED_DIM // L)]

                def so_body(so, lv):
                    r = se6 * NUM_STATES + so
                    for d in range(EMBED_DIM // L):
                        bbuf[q, r, pl.ds(d * L, L)] = lv[d]
                        bbuf[q, r, pl.ds(EMBED_DIM + d * L, L)] = (
                            lrtmp[NUM_STATES + so, pl.ds(d * L, L)]
                        )
                    return lv

                lax.fori_loop(0, NUM_STATES, so_body, lvals)
                return c2

            lax.fori_loop(0, NUM_STATES // 2, se_body, 0)
            pltpu.async_copy(
                bbuf.at[q],
                t_hbm.at[pl.ds(
                    cid * T_ROWS + (sid * PP_PER_TILE + ppl) * NPAIR_STATES
                    + q * HSLAB,
                    HSLAB)],
                bwsem,
            )
        return carry

    lax.fori_loop(0, PP_PER_TILE, build_pair, 0)
    for q in range(2):
        bwsem = wsem0 if q == 0 else wsem1
        pltpu.make_async_copy(
            bbuf.at[q], t_hbm.at[pl.ds(0, HSLAB)], bwsem
        ).wait()

    def pvec_body(k, carry):
        i16 = lax.iota(jnp.int32, L)
        pvec[pl.ds(k * L, L)] = (
            (lax.shift_right_logical(i16, 3) + 2 * k) * NPAIR_STATES
            + cid * T_ROWS
        )
        return carry

    lax.fori_loop(0, (N_PP * GRP) // L, pvec_body, 0)

    plsc.subcore_barrier()

    K_PER_UNIT = (N_PP * GRP) // L // N_UNITS

    def make_pidx_chunk(slx, u):
        slv = jnp.full((L,), slx, dtype=jnp.int32)

        def pidx_body(k, c2):
            i16 = lax.iota(jnp.int32, L)
            rvec = lax.bitwise_and(i16, 7)
            cvec = (lax.shift_right_logical(i16, 3) + 2 * k) * 2
            even = plsc.load_gather(ibuf, [slv, rvec, cvec])
            odd = plsc.load_gather(ibuf, [slv, rvec, cvec + 1])
            pibuf[slx, pl.ds(k * L, L)] = (
                even * NUM_STATES + odd + pvec[pl.ds(k * L, L)]
            )
            return c2

        return lax.fori_loop(u * K_PER_UNIT, (u + 1) * K_PER_UNIT, pidx_body, 0)

    g0 = wid * G_PER_W
    pltpu.sync_copy(grid_hbm.at[pl.ds(g0 * GRP, GRP)], ibuf.at[0])
    for u in range(N_UNITS):
        make_pidx_chunk(0, u)
    pltpu.async_copy(grid_hbm.at[pl.ds((g0 + 1) * GRP, GRP)], ibuf.at[1], isem)

    def group_body(gi, carry):
        g = g0 + gi
        sl = lax.rem(gi, 2)

        gsems = [gsem0, gsem1, gsem2, gsem3]
        wsems = [wsem0, wsem1, wsem2, wsem3]
        for u in range(N_UNITS):
            us = u % 4
            gsem = gsems[us]
            wsem = wsems[us]

            if u >= 4:
                pltpu.make_async_copy(
                    rbuf.at[us], out_hbm.at[g, pl.ds(u * UNIT, UNIT)], wsem
                ).wait()
            else:
                @pl.when(gi > 0)
                def _wait_prev_write():
                    pltpu.make_async_copy(
                        rbuf.at[us], out_hbm.at[g, pl.ds(u * UNIT, UNIT)], wsem
                    ).wait()

            gathers = [
                pltpu.async_copy(
                    t_hbm.at[pibuf.at[sl, pl.ds(u * UNIT + j * CHUNK, CHUNK)]],
                    rbuf.at[us, pl.ds(j * CHUNK, CHUNK)],
                    gsem,
                )
                for j in range(UNIT // CHUNK)
            ]

            @pl.when(gi < G_PER_W - 1)
            def _shadow_work():
                if u == 0:
                    pltpu.make_async_copy(
                        grid_hbm.at[pl.ds((g + 1) * GRP, GRP)],
                        ibuf.at[1 - sl], isem,
                    ).wait()

                    @pl.when(gi < G_PER_W - 2)
                    def _stage_next_idx():
                        pltpu.async_copy(
                            grid_hbm.at[pl.ds((g + 2) * GRP, GRP)],
                            ibuf.at[sl], isem,
                        )
                make_pidx_chunk(1 - sl, u)

            for gg in gathers:
                gg.wait()
            pltpu.async_copy(
                rbuf.at[us], out_hbm.at[g, pl.ds(u * UNIT, UNIT)], wsem
            )
        return carry

    lax.fori_loop(0, G_PER_W, group_body, 0)

    lastg = wid * G_PER_W + G_PER_W - 1
    for us, wsem in enumerate([wsem0, wsem1, wsem2, wsem3]):
        pltpu.make_async_copy(
            rbuf.at[us],
            out_hbm.at[lastg, pl.ds((N_UNITS - 4 + us) * UNIT, UNIT)],
            wsem,
        ).wait()


def kernel(grid_obs, state_table, pos_table):
    out4, _ = _sc_embed(grid_obs.astype(jnp.int32), state_table, pos_table)
    out = (
        out4.reshape(N_GRP, N_PP, GRP, ROW)
        .transpose(0, 2, 1, 3)
        .reshape(BATCH, N_POS * EMBED_DIM)
    )
    return out


# speedup vs baseline: 1.2433x; 1.2433x over previous
import functools

import jax
import jax.numpy as jnp
from jax import lax
from jax.experimental import pallas as pl
from jax.experimental.pallas import tpu as pltpu
from jax.experimental.pallas import tpu_sc as plsc

NUM_STATES = 12
NPAIR_STATES = NUM_STATES * NUM_STATES
EMBED_DIM = 64
ROW = 2 * EMBED_DIM
N_POS = 1024
N_PP = N_POS // 2
BATCH = 4096
GRP = 8
N_GRP = BATCH // GRP

NC = 2
NS = 16
L = 16
NW = NC * NS
G_PER_W = N_GRP // NW
CHUNK = 128
UNIT = 256
N_UNITS = (N_PP * GRP) // UNIT
T_ROWS = N_PP * NPAIR_STATES
PP_PER_TILE = N_PP // NS

_mesh = plsc.VectorSubcoreMesh(core_axis_name="c", subcore_axis_name="s")


@functools.partial(
    pl.kernel,
    out_type=[
        jax.ShapeDtypeStruct((N_GRP, N_PP * GRP, ROW), jnp.float32),
        jax.ShapeDtypeStruct((NC * T_ROWS, ROW), jnp.float32),
    ],
    mesh=_mesh,
    compiler_params=pltpu.CompilerParams(
        use_tc_tiling_on_sc=False, needs_layout_passes=False
    ),
    scratch_types=[
        pltpu.VMEM((NUM_STATES, EMBED_DIM), jnp.float32),
        pltpu.VMEM((2 * PP_PER_TILE, EMBED_DIM), jnp.float32),
        pltpu.VMEM((2, NPAIR_STATES // 2, ROW), jnp.float32),
        pltpu.VMEM((2 * NUM_STATES, EMBED_DIM), jnp.float32),
        pltpu.VMEM((N_PP * GRP,), jnp.int32),
        pltpu.VMEM((2, GRP, GRP, ROW), jnp.int32),
        pltpu.VMEM((2, N_PP * GRP), jnp.int32),
        pltpu.VMEM((2, UNIT, ROW), jnp.float32),
        pltpu.SemaphoreType.DMA,
        pltpu.SemaphoreType.DMA,
        pltpu.SemaphoreType.DMA,
        pltpu.SemaphoreType.DMA,
        pltpu.SemaphoreType.DMA,
    ],
)
def _sc_embed(grid_hbm, state_hbm, pos_hbm, out_hbm, t_hbm,
              sbuf, pbuf, bbuf, lrtmp, pvec, ibuf, pibuf, rbuf,
              gsem0, gsem1, isem, wsem0, wsem1):
    cid = lax.axis_index("c")
    sid = lax.axis_index("s")
    wid = sid * NC + cid

    pltpu.sync_copy(state_hbm, sbuf)
    pltpu.sync_copy(pos_hbm.at[pl.ds(sid * 2 * PP_PER_TILE, 2 * PP_PER_TILE)], pbuf)

    HSLAB = NPAIR_STATES // 2

    def build_pair(ppl, carry):
        def halves(s, c2):
            for d in range(EMBED_DIM // L):
                lrtmp[s, pl.ds(d * L, L)] = (
                    pbuf[2 * ppl, pl.ds(d * L, L)] + sbuf[s, pl.ds(d * L, L)]
                )
                lrtmp[NUM_STATES + s, pl.ds(d * L, L)] = (
                    pbuf[2 * ppl + 1, pl.ds(d * L, L)] + sbuf[s, pl.ds(d * L, L)]
                )
            return c2

        lax.fori_loop(0, NUM_STATES, halves, 0)

        for q in range(2):
            bwsem = wsem0 if q == 0 else wsem1

            @pl.when(ppl >= 1)
            def _wait_prev_slab():
                pltpu.make_async_copy(
                    bbuf.at[q], t_hbm.at[pl.ds(0, HSLAB)], bwsem
                ).wait()

            def se_body(se6, c2):
                se = se6 + q * (NUM_STATES // 2)
                lvals = [lrtmp[se, pl.ds(d * L, L)] for d in range(EMBED_DIM // L)]

                def so_body(so, lv):
                    r = se6 * NUM_STATES + so
                    for d in range(EMBED_DIM // L):
                        bbuf[q, r, pl.ds(d * L, L)] = lv[d]
                        bbuf[q, r, pl.ds(EMBED_DIM + d * L, L)] = (
                            lrtmp[NUM_STATES + so, pl.ds(d * L, L)]
                        )
                    return lv

                lax.fori_loop(0, NUM_STATES, so_body, lvals)
                return c2

            lax.fori_loop(0, NUM_STATES // 2, se_body, 0)
            pltpu.async_copy(
                bbuf.at[q],
                t_hbm.at[pl.ds(
                    cid * T_ROWS + (sid * PP_PER_TILE + ppl) * NPAIR_STATES
                    + q * HSLAB,
                    HSLAB)],
                bwsem,
            )
        return carry

    lax.fori_loop(0, PP_PER_TILE, build_pair, 0)
    for q in range(2):
        bwsem = wsem0 if q == 0 else wsem1
        pltpu.make_async_copy(
            bbuf.at[q], t_hbm.at[pl.ds(0, HSLAB)], bwsem
        ).wait()

    def pvec_body(k, carry):
        i16 = lax.iota(jnp.int32, L)
        pvec[pl.ds(k * L, L)] = (
            (lax.shift_right_logical(i16, 3) + 2 * k) * NPAIR_STATES
            + cid * T_ROWS
        )
        return carry

    lax.fori_loop(0, (N_PP * GRP) // L, pvec_body, 0)

    K_PER_UNIT = (N_PP * GRP) // L // N_UNITS

    def make_pidx_chunk(slx, u):
        slv = jnp.full((L,), slx, dtype=jnp.int32)

        def pidx_body(k, c2):
            i16 = lax.iota(jnp.int32, L)
            rvec = lax.bitwise_and(i16, 7)
            qvec = (lax.shift_right_logical(i16, 3) + 2 * k) * 2
            ctv = lax.shift_right_logical(qvec, 7)
            ccv = lax.bitwise_and(qvec, ROW - 1)
            even = plsc.load_gather(ibuf, [slv, ctv, rvec, ccv])
            odd = plsc.load_gather(ibuf, [slv, ctv, rvec, ccv + 1])
            pibuf[slx, pl.ds(k * L, L)] = (
                even * NUM_STATES + odd + pvec[pl.ds(k * L, L)]
            )
            return c2

        return lax.fori_loop(u * K_PER_UNIT, (u + 1) * K_PER_UNIT, pidx_body, 0)

    g0 = wid * G_PER_W
    pltpu.sync_copy(grid_hbm.at[g0], ibuf.at[0])
    for u in range(N_UNITS):
        make_pidx_chunk(0, u)
    pltpu.async_copy(grid_hbm.at[g0 + 1], ibuf.at[1], isem)

    plsc.subcore_barrier()

    def group_body(gi, carry):
        g = g0 + gi
        sl = lax.rem(gi, 2)

        for u in range(N_UNITS):
            us = u % 2
            gsem = gsem0 if us == 0 else gsem1
            wsem = wsem0 if us == 0 else wsem1

            if u >= 2:
                pltpu.make_async_copy(
                    rbuf.at[us], out_hbm.at[g, pl.ds(u * UNIT, UNIT)], wsem
                ).wait()
            else:
                @pl.when(gi > 0)
                def _wait_prev_write():
                    pltpu.make_async_copy(
                        rbuf.at[us], out_hbm.at[g, pl.ds(u * UNIT, UNIT)], wsem
                    ).wait()

            gathers = [
                pltpu.async_copy(
                    t_hbm.at[pibuf.at[sl, pl.ds(u * UNIT + j * CHUNK, CHUNK)]],
                    rbuf.at[us, pl.ds(j * CHUNK, CHUNK)],
                    gsem,
                )
                for j in range(UNIT // CHUNK)
            ]

            @pl.when(gi < G_PER_W - 1)
            def _shadow_work():
                if u == 0:
                    pltpu.make_async_copy(
                        grid_hbm.at[g + 1], ibuf.at[1 - sl], isem,
                    ).wait()

                    @pl.when(gi < G_PER_W - 2)
                    def _stage_next_idx():
                        pltpu.async_copy(
                            grid_hbm.at[g + 2], ibuf.at[sl], isem,
                        )
                make_pidx_chunk(1 - sl, u)

            for gg in gathers:
                gg.wait()
            pltpu.async_copy(
                rbuf.at[us], out_hbm.at[g, pl.ds(u * UNIT, UNIT)], wsem
            )
        return carry

    lax.fori_loop(0, G_PER_W, group_body, 0)

    lastg = wid * G_PER_W + G_PER_W - 1
    for us in range(2):
        wsem = wsem0 if us == 0 else wsem1
        pltpu.make_async_copy(
            rbuf.at[us],
            out_hbm.at[lastg, pl.ds((N_UNITS - 2 + us) * UNIT, UNIT)],
            wsem,
        ).wait()


def kernel(grid_obs, state_table, pos_table):
    grid4 = (
        grid_obs.astype(jnp.int32)
        .reshape(N_GRP, GRP, GRP, ROW)
        .transpose(0, 2, 1, 3)
    )
    out4, _ = _sc_embed(grid4, state_table, pos_table)
    out = (
        out4.reshape(N_GRP, N_PP, GRP, ROW)
        .transpose(0, 2, 1, 3)
        .reshape(BATCH, N_POS * EMBED_DIM)
    )
    return out
